# Initial kernel scaffold; baseline (speedup 1.0000x reference)
import jax
import jax.numpy as jnp
from jax.experimental import pallas as pl


def kernel(x, table, W1, b1, W2, b2):
    raise NotImplementedError("write your pallas kernel here")


# speedup vs baseline: 2.1497x; 2.1497x over previous
import functools

import jax
import jax.numpy as jnp
from jax import lax
from jax.experimental import pallas as pl
from jax.experimental.pallas import tpu as pltpu
from jax.experimental.pallas import tpu_sc as plsc

VOCAB = 1000000
EMBED_DIM = 32
HIDDEN_DIM = 64
OUTPUT_SIZE = 32
BATCH = 16384
HIST = 50

NUM_CORES = 2
NUM_SUBCORES = 16
NUM_WORKERS = NUM_CORES * NUM_SUBCORES

ROWS_W = BATCH // NUM_WORKERS
IDX_W = ROWS_W * HIST
GROUPS_PER_CHUNK = 8
CHUNK = GROUPS_PER_CHUNK * HIST
NCHUNK = IDX_W // CHUNK
SUB = 80
NSUB = CHUNK // SUB
LANES = 16


def _sc_gather_pool(x_flat, table):
    mesh = plsc.VectorSubcoreMesh(core_axis_name="c", subcore_axis_name="s")

    @functools.partial(
        pl.kernel,
        out_type=jax.ShapeDtypeStruct((BATCH, EMBED_DIM), jnp.float32),
        mesh=mesh,
        compiler_params=pltpu.CompilerParams(use_tc_tiling_on_sc=False),
        scratch_types=[
            pltpu.VMEM((IDX_W,), jnp.int32),
            pltpu.VMEM((CHUNK, EMBED_DIM), jnp.float32),
            pltpu.VMEM((ROWS_W, EMBED_DIM), jnp.float32),
        ],
    )
    def sc_kernel(x_hbm, table_hbm, out_hbm, idx_v, rows_v, pooled_v):
        wid = lax.axis_index("s") * NUM_CORES + lax.axis_index("c")
        pltpu.sync_copy(x_hbm.at[pl.ds(wid * IDX_W, IDX_W)], idx_v)

        @pl.loop(0, NCHUNK)
        def _(c):
            ibase = c * CHUNK
            for k in range(NSUB):
                pltpu.sync_copy(
                    table_hbm.at[idx_v.at[pl.ds(ibase + k * SUB, SUB)]],
                    rows_v.at[pl.ds(k * SUB, SUB)],
                )

            @pl.loop(0, GROUPS_PER_CHUNK)
            def _(g):
                rbase = g * HIST
                for k in range(2):
                    sl = pl.ds(k * LANES, LANES)
                    acc = rows_v[rbase, sl]
                    for j in range(1, HIST):
                        acc = acc + rows_v[rbase + j, sl]
                    pooled_v[c * GROUPS_PER_CHUNK + g, sl] = acc

        pltpu.sync_copy(pooled_v, out_hbm.at[pl.ds(wid * ROWS_W, ROWS_W)])

    return sc_kernel(x_flat, table)


def _mlp_body(p_ref, w1t_ref, b1_ref, w2t_ref, b2_ref, o_ref):
    p = p_ref[...] * jnp.float32(1.0 / HIST)
    h = jnp.dot(p, w1t_ref[...], preferred_element_type=jnp.float32)
    h = jnp.maximum(h + b1_ref[...], 0.0)
    o = jnp.dot(h, w2t_ref[...], preferred_element_type=jnp.float32)
    o_ref[...] = o + b2_ref[...]


def _tc_mlp(pooled_sums, W1, b1, W2, b2):
    blk = 2048
    grid = (BATCH // blk,)
    return pl.pallas_call(
        _mlp_body,
        grid=grid,
        in_specs=[
            pl.BlockSpec((blk, EMBED_DIM), lambda i: (i, 0)),
            pl.BlockSpec((EMBED_DIM, HIDDEN_DIM), lambda i: (0, 0)),
            pl.BlockSpec((1, HIDDEN_DIM), lambda i: (0, 0)),
            pl.BlockSpec((HIDDEN_DIM, OUTPUT_SIZE), lambda i: (0, 0)),
            pl.BlockSpec((1, OUTPUT_SIZE), lambda i: (0, 0)),
        ],
        out_specs=pl.BlockSpec((blk, OUTPUT_SIZE), lambda i: (i, 0)),
        out_shape=jax.ShapeDtypeStruct((BATCH, OUTPUT_SIZE), jnp.float32),
    )(
        pooled_sums,
        W1.T,
        b1.reshape(1, HIDDEN_DIM),
        W2.T,
        b2.reshape(1, OUTPUT_SIZE),
    )


def kernel(x, table, W1, b1, W2, b2):
    sums = _sc_gather_pool(x.reshape(-1), table)
    return _tc_mlp(sums, W1, b1, W2, b2)
